# Initial kernel scaffold; baseline (speedup 1.0000x reference)
import functools

import jax
import jax.numpy as jnp
from jax import lax
from jax.experimental import pallas as pl
from jax.experimental.pallas import tpu as pltpu
from jax.experimental.pallas import tpu_sc as plsc

N = 10000
E = 320000
NC = 2
NS = 16
NW = NC * NS
EDGES_PER_W = E // NW
CHUNK = 80
NCHUNK = EDGES_PER_W // CHUNK
ROWS_PER_TILE = N // NS
DEGW = 8

_mesh = lambda: plsc.VectorSubcoreMesh(core_axis_name="c", subcore_axis_name="s")


@functools.lru_cache(maxsize=None)
def _make_sc_pass(w):
    @functools.partial(
        pl.kernel,
        out_type=jax.ShapeDtypeStruct((NC, N, w), jnp.float32),
        mesh=_mesh(),
        scratch_types=[
            pltpu.VMEM((CHUNK,), jnp.int32),
            pltpu.VMEM((CHUNK,), jnp.int32),
            pltpu.VMEM((CHUNK, w), jnp.float32),
            pltpu.VMEM_SHARED((N, w), jnp.float32),
            pltpu.SemaphoreType.DMA,
        ],
    )
    def sc_pass(g_hbm, src_hbm, dst_hbm, zeros_hbm, out_hbm,
                src_v, dst_v, msg_v, acc_sh, sem):
        c = lax.axis_index("c")
        s = lax.axis_index("s")
        r0 = s * ROWS_PER_TILE

        @pl.when(c == 0)
        def _():
            pltpu.sync_copy(g_hbm.at[pl.ds(r0, ROWS_PER_TILE)],
                            acc_sh.at[pl.ds(r0, ROWS_PER_TILE)])

        @pl.when(c != 0)
        def _():
            pltpu.sync_copy(zeros_hbm.at[pl.ds(r0, ROWS_PER_TILE)],
                            acc_sh.at[pl.ds(r0, ROWS_PER_TILE)])

        plsc.subcore_barrier()

        base = (s * NC + c) * EDGES_PER_W

        @pl.loop(0, NCHUNK)
        def _(i):
            off = base + i * CHUNK
            pltpu.sync_copy(src_hbm.at[pl.ds(off, CHUNK)], src_v)
            pltpu.sync_copy(dst_hbm.at[pl.ds(off, CHUNK)], dst_v)
            pltpu.async_copy(g_hbm.at[src_v], msg_v, sem).wait()
            pltpu.sync_copy(msg_v, acc_sh.at[dst_v], add=True)

        plsc.subcore_barrier()
        pltpu.sync_copy(acc_sh.at[pl.ds(r0, ROWS_PER_TILE)],
                        out_hbm.at[c].at[pl.ds(r0, ROWS_PER_TILE)])

    return sc_pass


def _make_deg_pass():
    @functools.partial(
        pl.kernel,
        out_type=jax.ShapeDtypeStruct((NC, N, DEGW), jnp.float32),
        mesh=_mesh(),
        scratch_types=[
            pltpu.VMEM((CHUNK,), jnp.int32),
            pltpu.VMEM((CHUNK, DEGW), jnp.float32),
            pltpu.VMEM_SHARED((N, DEGW), jnp.float32),
        ],
    )
    def deg_pass(dst_hbm, ones_hbm, zeros_hbm, out_hbm,
                 dst_v, ones_v, acc_sh):
        c = lax.axis_index("c")
        s = lax.axis_index("s")
        r0 = s * ROWS_PER_TILE
        pltpu.sync_copy(zeros_hbm.at[pl.ds(r0, ROWS_PER_TILE)],
                        acc_sh.at[pl.ds(r0, ROWS_PER_TILE)])
        pltpu.sync_copy(ones_hbm, ones_v)
        plsc.subcore_barrier()

        base = (s * NC + c) * EDGES_PER_W

        @pl.loop(0, NCHUNK)
        def _(i):
            off = base + i * CHUNK
            pltpu.sync_copy(dst_hbm.at[pl.ds(off, CHUNK)], dst_v)
            pltpu.sync_copy(ones_v, acc_sh.at[dst_v], add=True)

        plsc.subcore_barrier()
        pltpu.sync_copy(acc_sh.at[pl.ds(r0, ROWS_PER_TILE)],
                        out_hbm.at[c].at[pl.ds(r0, ROWS_PER_TILE)])

    return deg_pass


_deg_pass = _make_deg_pass()


_R = 1000
_GRID = N // _R


def _rows(shape_tail):
    ntail = len(shape_tail)
    return pl.BlockSpec((_R,) + shape_tail, lambda i: (i,) + (0,) * ntail)


def _full(shape):
    nd = len(shape)
    return pl.BlockSpec(shape, lambda i: (0,) * nd)


def _acc_spec(w):
    return pl.BlockSpec((NC, _R, w), lambda i: (0, i, 0))


def _tc_call(body, in_specs, out_specs, out_shape):
    return pl.pallas_call(
        body,
        grid=(_GRID,),
        in_specs=in_specs,
        out_specs=out_specs,
        out_shape=out_shape,
    )


def _tc_a(degp, x, W1):
    def body(degp_ref, x_ref, w1_ref, dinv_ref, g1_ref):
        deg = degp_ref[0] + degp_ref[1] + 1.0
        dinv8 = lax.rsqrt(deg)
        dinv = dinv8[:, 0:1]
        dinv_ref[...] = dinv
        h = jnp.dot(x_ref[...], w1_ref[...], preferred_element_type=jnp.float32)
        g1_ref[...] = h * dinv

    return _tc_call(
        body,
        [_acc_spec(DEGW), _rows((128,)), _full((128, 64))],
        [_rows((1,)), _rows((64,))],
        [jax.ShapeDtypeStruct((N, 1), jnp.float32),
         jax.ShapeDtypeStruct((N, 64), jnp.float32)],
    )(degp, x, W1)


def _tc_b(acc1, g1, dinv, b1):
    def body(acc_ref, g_ref, dinv_ref, b_ref, out_ref):
        dinv = dinv_ref[...]
        t = (acc_ref[0] + acc_ref[1] + g_ref[...]) * dinv + b_ref[...]
        out_ref[...] = jnp.maximum(t, 0.0) * dinv

    return _tc_call(
        body,
        [_acc_spec(64), _rows((64,)), _rows((1,)), _full((1, 64))],
        _rows((64,)),
        jax.ShapeDtypeStruct((N, 64), jnp.float32),
    )(acc1, g1, dinv, b1.reshape(1, -1))


def _tc_c(acc2, g2, dinv, W2, b2, W3):
    def body(acc_ref, g_ref, dinv_ref, w2_ref, b2_ref, w3_ref, out_ref):
        dinv = dinv_ref[...]
        s2 = (acc_ref[0] + acc_ref[1] + g_ref[...]) * dinv
        x3 = jnp.maximum(
            jnp.dot(s2, w2_ref[...], preferred_element_type=jnp.float32)
            + b2_ref[...], 0.0)
        h3 = jnp.dot(x3, w3_ref[...], preferred_element_type=jnp.float32)
        out_ref[...] = h3 * dinv

    return _tc_call(
        body,
        [_acc_spec(64), _rows((64,)), _rows((1,)),
         _full((64, 128)), _full((1, 128)), _full((128, 64))],
        _rows((64,)),
        jax.ShapeDtypeStruct((N, 64), jnp.float32),
    )(acc2, g2, dinv, W2, b2.reshape(1, -1), W3)


def _tc_d(acc3, g3, dinv, b3, W4):
    def body(acc_ref, g_ref, dinv_ref, b_ref, w4_ref, out_ref):
        dinv = dinv_ref[...]
        x4 = jnp.maximum((acc_ref[0] + acc_ref[1] + g_ref[...]) * dinv
                         + b_ref[...], 0.0)
        h4 = jnp.dot(x4, w4_ref[...], preferred_element_type=jnp.float32)
        out_ref[...] = h4 * dinv

    return _tc_call(
        body,
        [_acc_spec(64), _rows((64,)), _rows((1,)), _full((1, 64)),
         _full((64, 16))],
        _rows((16,)),
        jax.ShapeDtypeStruct((N, 16), jnp.float32),
    )(acc3, g3, dinv, b3.reshape(1, -1), W4)


def _tc_e(acc4, g4, dinv, b4):
    def body(acc_ref, g_ref, dinv_ref, b_ref, out_ref):
        dinv = dinv_ref[...]
        x5 = jnp.maximum((acc_ref[0] + acc_ref[1] + g_ref[...]) * dinv
                         + b_ref[...], 0.0)
        out_ref[...] = x5 * dinv

    return _tc_call(
        body,
        [_acc_spec(16), _rows((16,)), _rows((1,)), _full((1, 16))],
        _rows((16,)),
        jax.ShapeDtypeStruct((N, 16), jnp.float32),
    )(acc4, g4, dinv, b4.reshape(1, -1))


def _tc_f(acc5, g5, dinv, W5, b5):
    def body(acc_ref, g_ref, dinv_ref, w5_ref, b5_ref, out_ref):
        dinv = dinv_ref[...]
        s5 = (acc_ref[0] + acc_ref[1] + g_ref[...]) * dinv
        z = jnp.dot(s5, w5_ref[...], preferred_element_type=jnp.float32) + b5_ref[...]
        m = jnp.max(z, axis=1, keepdims=True)
        lse = m + jnp.log(jnp.sum(jnp.exp(z - m), axis=1, keepdims=True))
        out_ref[...] = z - lse

    return _tc_call(
        body,
        [_acc_spec(16), _rows((16,)), _rows((1,)),
         _full((16, 2)), _full((1, 2))],
        _rows((2,)),
        jax.ShapeDtypeStruct((N, 2), jnp.float32),
    )(acc5, g5, dinv, W5, b5.reshape(1, -1))


def kernel(x, edge_index, W1, b1, W2, b2, W3, b3, W4, b4, W5, b5):
    src = edge_index[0]
    dst = edge_index[1]
    zeros64 = jnp.zeros((N, 64), jnp.float32)
    zeros16 = jnp.zeros((N, 16), jnp.float32)
    zeros8 = jnp.zeros((N, DEGW), jnp.float32)
    ones = jnp.ones((CHUNK, DEGW), jnp.float32)

    sc64 = _make_sc_pass(64)
    sc16 = _make_sc_pass(16)

    degp = _deg_pass(dst, ones, zeros8)
    dinv, g1 = _tc_a(degp, x, W1)

    acc1 = sc64(g1, src, dst, zeros64)
    g2 = _tc_b(acc1, g1, dinv, b1)

    acc2 = sc64(g2, src, dst, zeros64)
    g3 = _tc_c(acc2, g2, dinv, W2, b2, W3)

    acc3 = sc64(g3, src, dst, zeros64)
    g4 = _tc_d(acc3, g3, dinv, b3, W4)

    acc4 = sc16(g4, src, dst, zeros16)
    g5 = _tc_e(acc4, g4, dinv, b4)

    acc5 = sc16(g5, src, dst, zeros16)
    return _tc_f(acc5, g5, dinv, W5, b5)


# speedup vs baseline: 11.2263x; 11.2263x over previous
import functools

import jax
import jax.numpy as jnp
from jax import lax
from jax.experimental import pallas as pl
from jax.experimental.pallas import tpu as pltpu
from jax.experimental.pallas import tpu_sc as plsc

N = 10000
NP = 10240
E = 320000
NC = 2
NS = 16
NW = NC * NS
EDGES_PER_W = E // NW
CHUNK = 80
NCHUNK = EDGES_PER_W // CHUNK
ROWS_PER_TILE = NP // NS
DEGW = 8

_mesh = lambda: plsc.VectorSubcoreMesh(core_axis_name="c", subcore_axis_name="s")


@functools.lru_cache(maxsize=None)
def _make_sc_pass(w):
    @functools.partial(
        pl.kernel,
        out_type=jax.ShapeDtypeStruct((NC, NP, w), jnp.float32),
        mesh=_mesh(),
        compiler_params=pltpu.CompilerParams(use_tc_tiling_on_sc=False),
        scratch_types=[
            pltpu.VMEM((CHUNK,), jnp.int32),
            pltpu.VMEM((CHUNK,), jnp.int32),
            pltpu.VMEM((CHUNK, w), jnp.float32),
            pltpu.VMEM_SHARED((NP, w), jnp.float32),
            pltpu.SemaphoreType.DMA,
        ],
    )
    def sc_pass(g_hbm, src_hbm, dst_hbm, zeros_hbm, out_hbm,
                src_v, dst_v, msg_v, acc_sh, sem):
        c = lax.axis_index("c")
        s = lax.axis_index("s")
        r0 = s * ROWS_PER_TILE

        @pl.when(c == 0)
        def _():
            pltpu.sync_copy(g_hbm.at[pl.ds(r0, ROWS_PER_TILE)],
                            acc_sh.at[pl.ds(r0, ROWS_PER_TILE)])

        @pl.when(c != 0)
        def _():
            pltpu.sync_copy(zeros_hbm.at[pl.ds(r0, ROWS_PER_TILE)],
                            acc_sh.at[pl.ds(r0, ROWS_PER_TILE)])

        plsc.subcore_barrier()

        base = (s * NC + c) * EDGES_PER_W

        @pl.loop(0, NCHUNK)
        def _(i):
            off = base + i * CHUNK
            pltpu.sync_copy(src_hbm.at[pl.ds(off, CHUNK)], src_v)
            pltpu.sync_copy(dst_hbm.at[pl.ds(off, CHUNK)], dst_v)
            pltpu.async_copy(g_hbm.at[src_v], msg_v, sem).wait()
            pltpu.sync_copy(msg_v, acc_sh.at[dst_v], add=True)

        plsc.subcore_barrier()
        pltpu.sync_copy(acc_sh.at[pl.ds(r0, ROWS_PER_TILE)],
                        out_hbm.at[c].at[pl.ds(r0, ROWS_PER_TILE)])

    return sc_pass


@functools.lru_cache(maxsize=None)
def _make_deg_pass():
    @functools.partial(
        pl.kernel,
        out_type=jax.ShapeDtypeStruct((NC, NP, DEGW), jnp.float32),
        mesh=_mesh(),
        compiler_params=pltpu.CompilerParams(use_tc_tiling_on_sc=False),
        scratch_types=[
            pltpu.VMEM((CHUNK,), jnp.int32),
            pltpu.VMEM((CHUNK, DEGW), jnp.float32),
            pltpu.VMEM_SHARED((NP, DEGW), jnp.float32),
        ],
    )
    def deg_pass(dst_hbm, ones_hbm, zeros_hbm, out_hbm,
                 dst_v, ones_v, acc_sh):
        c = lax.axis_index("c")
        s = lax.axis_index("s")
        r0 = s * ROWS_PER_TILE
        pltpu.sync_copy(zeros_hbm.at[pl.ds(r0, ROWS_PER_TILE)],
                        acc_sh.at[pl.ds(r0, ROWS_PER_TILE)])
        pltpu.sync_copy(ones_hbm, ones_v)
        plsc.subcore_barrier()

        base = (s * NC + c) * EDGES_PER_W

        @pl.loop(0, NCHUNK)
        def _(i):
            off = base + i * CHUNK
            pltpu.sync_copy(dst_hbm.at[pl.ds(off, CHUNK)], dst_v)
            pltpu.sync_copy(ones_v, acc_sh.at[dst_v], add=True)

        plsc.subcore_barrier()
        pltpu.sync_copy(acc_sh.at[pl.ds(r0, ROWS_PER_TILE)],
                        out_hbm.at[c].at[pl.ds(r0, ROWS_PER_TILE)])

    return deg_pass


_R = 640
_GRID = NP // _R


def _rows(shape_tail):
    ntail = len(shape_tail)
    return pl.BlockSpec((_R,) + shape_tail, lambda i: (i,) + (0,) * ntail)


def _full(shape):
    nd = len(shape)
    return pl.BlockSpec(shape, lambda i: (0,) * nd)


def _acc_spec(w):
    return pl.BlockSpec((NC, _R, w), lambda i: (0, i, 0))


def _tc_call(body, in_specs, out_specs, out_shape):
    return pl.pallas_call(
        body,
        grid=(_GRID,),
        in_specs=in_specs,
        out_specs=out_specs,
        out_shape=out_shape,
    )


def _tc_a(degp, x, W1):
    def body(degp_ref, x_ref, w1_ref, dinv_ref, g1_ref):
        deg = degp_ref[0] + degp_ref[1] + 1.0
        dinv8 = lax.rsqrt(deg)
        dinv = dinv8[:, 0:1]
        dinv_ref[...] = dinv
        h = jnp.dot(x_ref[...], w1_ref[...], preferred_element_type=jnp.float32)
        g1_ref[...] = h * dinv

    return _tc_call(
        body,
        [_acc_spec(DEGW), _rows((128,)), _full((128, 64))],
        [_rows((1,)), _rows((64,))],
        [jax.ShapeDtypeStruct((NP, 1), jnp.float32),
         jax.ShapeDtypeStruct((NP, 64), jnp.float32)],
    )(degp, x, W1)


def _tc_b(acc1, dinv, b1):
    def body(acc_ref, dinv_ref, b_ref, out_ref):
        dinv = dinv_ref[...]
        t = (acc_ref[0] + acc_ref[1]) * dinv + b_ref[...]
        out_ref[...] = jnp.maximum(t, 0.0) * dinv

    return _tc_call(
        body,
        [_acc_spec(64), _rows((1,)), _full((1, 64))],
        _rows((64,)),
        jax.ShapeDtypeStruct((NP, 64), jnp.float32),
    )(acc1, dinv, b1.reshape(1, -1))


def _tc_c(acc2, dinv, W2, b2, W3):
    def body(acc_ref, dinv_ref, w2_ref, b2_ref, w3_ref, out_ref):
        dinv = dinv_ref[...]
        s2 = (acc_ref[0] + acc_ref[1]) * dinv
        x3 = jnp.maximum(
            jnp.dot(s2, w2_ref[...], preferred_element_type=jnp.float32)
            + b2_ref[...], 0.0)
        h3 = jnp.dot(x3, w3_ref[...], preferred_element_type=jnp.float32)
        out_ref[...] = h3 * dinv

    return _tc_call(
        body,
        [_acc_spec(64), _rows((1,)),
         _full((64, 128)), _full((1, 128)), _full((128, 64))],
        _rows((64,)),
        jax.ShapeDtypeStruct((NP, 64), jnp.float32),
    )(acc2, dinv, W2, b2.reshape(1, -1), W3)


def _tc_d(acc3, dinv, b3, W4):
    def body(acc_ref, dinv_ref, b_ref, w4_ref, out_ref):
        dinv = dinv_ref[...]
        x4 = jnp.maximum((acc_ref[0] + acc_ref[1]) * dinv
                         + b_ref[...], 0.0)
        h4 = jnp.dot(x4, w4_ref[...], preferred_element_type=jnp.float32)
        out_ref[...] = h4 * dinv

    return _tc_call(
        body,
        [_acc_spec(64), _rows((1,)), _full((1, 64)),
         _full((64, 16))],
        _rows((16,)),
        jax.ShapeDtypeStruct((NP, 16), jnp.float32),
    )(acc3, dinv, b3.reshape(1, -1), W4)


def _tc_e(acc4, dinv, b4):
    def body(acc_ref, dinv_ref, b_ref, out_ref):
        dinv = dinv_ref[...]
        x5 = jnp.maximum((acc_ref[0] + acc_ref[1]) * dinv
                         + b_ref[...], 0.0)
        out_ref[...] = x5 * dinv

    return _tc_call(
        body,
        [_acc_spec(16), _rows((1,)), _full((1, 16))],
        _rows((16,)),
        jax.ShapeDtypeStruct((NP, 16), jnp.float32),
    )(acc4, dinv, b4.reshape(1, -1))


def _tc_f(acc5, dinv, W5, b5):
    def body(acc_ref, dinv_ref, w5_ref, b5_ref, out_ref):
        dinv = dinv_ref[...]
        s5 = (acc_ref[0] + acc_ref[1]) * dinv
        z = jnp.dot(s5, w5_ref[...], preferred_element_type=jnp.float32) + b5_ref[...]
        m = jnp.max(z, axis=1, keepdims=True)
        lse = m + jnp.log(jnp.sum(jnp.exp(z - m), axis=1, keepdims=True))
        out_ref[...] = z - lse

    return _tc_call(
        body,
        [_acc_spec(16), _rows((1,)),
         _full((16, 2)), _full((1, 2))],
        _rows((2,)),
        jax.ShapeDtypeStruct((NP, 2), jnp.float32),
    )(acc5, dinv, W5, b5.reshape(1, -1))


def kernel(x, edge_index, W1, b1, W2, b2, W3, b3, W4, b4, W5, b5):
    src = edge_index[0]
    dst = edge_index[1]
    x = jnp.pad(x, ((0, NP - N), (0, 0)))
    zeros64 = jnp.zeros((NP, 64), jnp.float32)
    zeros16 = jnp.zeros((NP, 16), jnp.float32)
    zeros8 = jnp.zeros((NP, DEGW), jnp.float32)
    ones = jnp.ones((CHUNK, DEGW), jnp.float32)

    sc64 = _make_sc_pass(64)
    sc16 = _make_sc_pass(16)

    degp = _make_deg_pass()(dst, ones, zeros8)
    dinv, g1 = _tc_a(degp, x, W1)

    acc1 = sc64(g1, src, dst, zeros64)
    g2 = _tc_b(acc1, dinv, b1)

    acc2 = sc64(g2, src, dst, zeros64)
    g3 = _tc_c(acc2, dinv, W2, b2, W3)

    acc3 = sc64(g3, src, dst, zeros64)
    g4 = _tc_d(acc3, dinv, b3, W4)

    acc4 = sc16(g4, src, dst, zeros16)
    g5 = _tc_e(acc4, dinv, b4)

    acc5 = sc16(g5, src, dst, zeros16)
    return _tc_f(acc5, dinv, W5, b5)[:N]


# speedup vs baseline: 30.0762x; 2.6791x over previous
import functools

import jax
import jax.numpy as jnp
from jax import lax
from jax.experimental import pallas as pl
from jax.experimental.pallas import tpu as pltpu
from jax.experimental.pallas import tpu_sc as plsc

N = 10000
NP = 10240
E = 320000
NC = 2
NS = 16
NW = NC * NS
EDGES_PER_W = E // NW
CHUNK = 80
NCHUNK = EDGES_PER_W // CHUNK
NBUF = 5
NGROUP = NCHUNK // NBUF
ROWS_PER_TILE = NP // NS
DEGW = 8

_mesh = lambda: plsc.VectorSubcoreMesh(core_axis_name="c", subcore_axis_name="s")


@functools.lru_cache(maxsize=None)
def _make_sc_pass(w):
    @functools.partial(
        pl.kernel,
        out_type=jax.ShapeDtypeStruct((NC, NP, w), jnp.float32),
        mesh=_mesh(),
        compiler_params=pltpu.CompilerParams(use_tc_tiling_on_sc=False),
        scratch_types=[
            pltpu.VMEM((NCHUNK, CHUNK), jnp.int32),
            pltpu.VMEM((NCHUNK, CHUNK), jnp.int32),
            pltpu.VMEM((NBUF, CHUNK, w), jnp.float32),
            pltpu.VMEM_SHARED((NP, w), jnp.float32),
            pltpu.SemaphoreType.DMA,
            pltpu.SemaphoreType.DMA,
        ],
    )
    def sc_pass(g_hbm, src_hbm, dst_hbm, zeros_hbm, out_hbm,
                srcs_v, dsts_v, msg_v, acc_sh, gsem, ssem):
        c = lax.axis_index("c")
        s = lax.axis_index("s")
        r0 = s * ROWS_PER_TILE
        wid = s * NC + c

        isem = gsem
        idesc = pltpu.async_copy(src_hbm.at[wid], srcs_v, isem)
        idesc2 = pltpu.async_copy(dst_hbm.at[wid], dsts_v, isem)

        @pl.when(c == 0)
        def _():
            pltpu.sync_copy(g_hbm.at[pl.ds(r0, ROWS_PER_TILE)],
                            acc_sh.at[pl.ds(r0, ROWS_PER_TILE)])

        @pl.when(c != 0)
        def _():
            pltpu.sync_copy(zeros_hbm.at[pl.ds(r0, ROWS_PER_TILE)],
                            acc_sh.at[pl.ds(r0, ROWS_PER_TILE)])

        idesc.wait()
        idesc2.wait()
        plsc.subcore_barrier()

        @pl.loop(0, NGROUP)
        def _(grp):
            base = grp * NBUF
            gds = [
                pltpu.async_copy(g_hbm.at[srcs_v.at[base + b]],
                                 msg_v.at[b], gsem)
                for b in range(NBUF)
            ]
            for d in gds:
                d.wait()
            sds = [
                pltpu.async_copy(msg_v.at[b],
                                 acc_sh.at[dsts_v.at[base + b]],
                                 ssem, add=True)
                for b in range(NBUF)
            ]
            for d in sds:
                d.wait()

        plsc.subcore_barrier()
        pltpu.sync_copy(acc_sh.at[pl.ds(r0, ROWS_PER_TILE)],
                        out_hbm.at[c].at[pl.ds(r0, ROWS_PER_TILE)])

    return sc_pass


@functools.lru_cache(maxsize=None)
def _make_deg_pass():
    @functools.partial(
        pl.kernel,
        out_type=jax.ShapeDtypeStruct((NC, NP, DEGW), jnp.float32),
        mesh=_mesh(),
        compiler_params=pltpu.CompilerParams(use_tc_tiling_on_sc=False),
        scratch_types=[
            pltpu.VMEM((NCHUNK, CHUNK), jnp.int32),
            pltpu.VMEM((CHUNK, DEGW), jnp.float32),
            pltpu.VMEM_SHARED((NP, DEGW), jnp.float32),
            pltpu.SemaphoreType.DMA,
        ],
    )
    def deg_pass(dst_hbm, ones_hbm, zeros_hbm, out_hbm,
                 dsts_v, ones_v, acc_sh, ssem):
        c = lax.axis_index("c")
        s = lax.axis_index("s")
        r0 = s * ROWS_PER_TILE
        wid = s * NC + c
        pltpu.sync_copy(dst_hbm.at[wid], dsts_v)
        pltpu.sync_copy(zeros_hbm.at[pl.ds(r0, ROWS_PER_TILE)],
                        acc_sh.at[pl.ds(r0, ROWS_PER_TILE)])
        pltpu.sync_copy(ones_hbm, ones_v)
        plsc.subcore_barrier()

        @pl.loop(0, NGROUP)
        def _(grp):
            base = grp * NBUF
            sds = [
                pltpu.async_copy(ones_v,
                                 acc_sh.at[dsts_v.at[base + b]],
                                 ssem, add=True)
                for b in range(NBUF)
            ]
            for d in sds:
                d.wait()

        plsc.subcore_barrier()
        pltpu.sync_copy(acc_sh.at[pl.ds(r0, ROWS_PER_TILE)],
                        out_hbm.at[c].at[pl.ds(r0, ROWS_PER_TILE)])

    return deg_pass


_R = 640
_GRID = NP // _R


def _rows(shape_tail):
    ntail = len(shape_tail)
    return pl.BlockSpec((_R,) + shape_tail, lambda i: (i,) + (0,) * ntail)


def _full(shape):
    nd = len(shape)
    return pl.BlockSpec(shape, lambda i: (0,) * nd)


def _acc_spec(w):
    return pl.BlockSpec((NC, _R, w), lambda i: (0, i, 0))


def _tc_call(body, in_specs, out_specs, out_shape):
    return pl.pallas_call(
        body,
        grid=(_GRID,),
        in_specs=in_specs,
        out_specs=out_specs,
        out_shape=out_shape,
    )


def _tc_a(degp, x, W1):
    def body(degp_ref, x_ref, w1_ref, dinv_ref, g1_ref):
        deg = degp_ref[0] + degp_ref[1] + 1.0
        dinv8 = lax.rsqrt(deg)
        dinv = dinv8[:, 0:1]
        dinv_ref[...] = dinv
        h = jnp.dot(x_ref[...], w1_ref[...], preferred_element_type=jnp.float32)
        g1_ref[...] = h * dinv

    return _tc_call(
        body,
        [_acc_spec(DEGW), _rows((128,)), _full((128, 64))],
        [_rows((1,)), _rows((64,))],
        [jax.ShapeDtypeStruct((NP, 1), jnp.float32),
         jax.ShapeDtypeStruct((NP, 64), jnp.float32)],
    )(degp, x, W1)


def _tc_b(acc1, dinv, b1):
    def body(acc_ref, dinv_ref, b_ref, out_ref):
        dinv = dinv_ref[...]
        t = (acc_ref[0] + acc_ref[1]) * dinv + b_ref[...]
        out_ref[...] = jnp.maximum(t, 0.0) * dinv

    return _tc_call(
        body,
        [_acc_spec(64), _rows((1,)), _full((1, 64))],
        _rows((64,)),
        jax.ShapeDtypeStruct((NP, 64), jnp.float32),
    )(acc1, dinv, b1.reshape(1, -1))


def _tc_c(acc2, dinv, W2, b2, W3):
    def body(acc_ref, dinv_ref, w2_ref, b2_ref, w3_ref, out_ref):
        dinv = dinv_ref[...]
        s2 = (acc_ref[0] + acc_ref[1]) * dinv
        x3 = jnp.maximum(
            jnp.dot(s2, w2_ref[...], preferred_element_type=jnp.float32)
            + b2_ref[...], 0.0)
        h3 = jnp.dot(x3, w3_ref[...], preferred_element_type=jnp.float32)
        out_ref[...] = h3 * dinv

    return _tc_call(
        body,
        [_acc_spec(64), _rows((1,)),
         _full((64, 128)), _full((1, 128)), _full((128, 64))],
        _rows((64,)),
        jax.ShapeDtypeStruct((NP, 64), jnp.float32),
    )(acc2, dinv, W2, b2.reshape(1, -1), W3)


def _tc_d(acc3, dinv, b3, W4):
    def body(acc_ref, dinv_ref, b_ref, w4_ref, out_ref):
        dinv = dinv_ref[...]
        x4 = jnp.maximum((acc_ref[0] + acc_ref[1]) * dinv
                         + b_ref[...], 0.0)
        h4 = jnp.dot(x4, w4_ref[...], preferred_element_type=jnp.float32)
        out_ref[...] = h4 * dinv

    return _tc_call(
        body,
        [_acc_spec(64), _rows((1,)), _full((1, 64)),
         _full((64, 16))],
        _rows((16,)),
        jax.ShapeDtypeStruct((NP, 16), jnp.float32),
    )(acc3, dinv, b3.reshape(1, -1), W4)


def _tc_e(acc4, dinv, b4):
    def body(acc_ref, dinv_ref, b_ref, out_ref):
        dinv = dinv_ref[...]
        x5 = jnp.maximum((acc_ref[0] + acc_ref[1]) * dinv
                         + b_ref[...], 0.0)
        out_ref[...] = x5 * dinv

    return _tc_call(
        body,
        [_acc_spec(16), _rows((1,)), _full((1, 16))],
        _rows((16,)),
        jax.ShapeDtypeStruct((NP, 16), jnp.float32),
    )(acc4, dinv, b4.reshape(1, -1))


def _tc_f(acc5, dinv, W5, b5):
    def body(acc_ref, dinv_ref, w5_ref, b5_ref, out_ref):
        dinv = dinv_ref[...]
        s5 = (acc_ref[0] + acc_ref[1]) * dinv
        z = jnp.dot(s5, w5_ref[...], preferred_element_type=jnp.float32) + b5_ref[...]
        m = jnp.max(z, axis=1, keepdims=True)
        lse = m + jnp.log(jnp.sum(jnp.exp(z - m), axis=1, keepdims=True))
        out_ref[...] = z - lse

    return _tc_call(
        body,
        [_acc_spec(16), _rows((1,)),
         _full((16, 2)), _full((1, 2))],
        _rows((2,)),
        jax.ShapeDtypeStruct((NP, 2), jnp.float32),
    )(acc5, dinv, W5, b5.reshape(1, -1))


def kernel(x, edge_index, W1, b1, W2, b2, W3, b3, W4, b4, W5, b5):
    src = edge_index[0].reshape(NW, NCHUNK, CHUNK)
    dst = edge_index[1].reshape(NW, NCHUNK, CHUNK)
    x = jnp.pad(x, ((0, NP - N), (0, 0)))
    zeros64 = jnp.zeros((NP, 64), jnp.float32)
    zeros16 = jnp.zeros((NP, 16), jnp.float32)
    zeros8 = jnp.zeros((NP, DEGW), jnp.float32)
    ones = jnp.ones((CHUNK, DEGW), jnp.float32)

    sc64 = _make_sc_pass(64)
    sc16 = _make_sc_pass(16)

    degp = _make_deg_pass()(dst, ones, zeros8)
    dinv, g1 = _tc_a(degp, x, W1)

    acc1 = sc64(g1, src, dst, zeros64)
    g2 = _tc_b(acc1, dinv, b1)

    acc2 = sc64(g2, src, dst, zeros64)
    g3 = _tc_c(acc2, dinv, W2, b2, W3)

    acc3 = sc64(g3, src, dst, zeros64)
    g4 = _tc_d(acc3, dinv, b3, W4)

    acc4 = sc16(g4, src, dst, zeros16)
    g5 = _tc_e(acc4, dinv, b4)

    acc5 = sc16(g5, src, dst, zeros16)
    return _tc_f(acc5, dinv, W5, b5)[:N]


# speedup vs baseline: 34.6437x; 1.1519x over previous
import functools

import jax
import jax.numpy as jnp
from jax import lax
from jax.experimental import pallas as pl
from jax.experimental.pallas import tpu as pltpu
from jax.experimental.pallas import tpu_sc as plsc

N = 10000
NP = 10240
E = 320000
NC = 2
NS = 16
NW = NC * NS
EDGES_PER_W = E // NW
CHUNK = 80
NCHUNK = EDGES_PER_W // CHUNK
NBUF = 5
NGROUP = NCHUNK // NBUF
ROWS_PER_TILE = NP // NS
DEGW = 8

_mesh = lambda: plsc.VectorSubcoreMesh(core_axis_name="c", subcore_axis_name="s")


@functools.lru_cache(maxsize=None)
def _make_sc_pass(w):
    @functools.partial(
        pl.kernel,
        out_type=jax.ShapeDtypeStruct((NC, NP, w), jnp.float32),
        mesh=_mesh(),
        compiler_params=pltpu.CompilerParams(use_tc_tiling_on_sc=False),
        scratch_types=[
            pltpu.VMEM((NCHUNK, CHUNK), jnp.int32),
            pltpu.VMEM((NCHUNK, CHUNK), jnp.int32),
            pltpu.VMEM((2, NBUF, CHUNK, w), jnp.float32),
            pltpu.VMEM_SHARED((NP, w), jnp.float32),
            pltpu.SemaphoreType.DMA,
            pltpu.SemaphoreType.DMA,
        ],
    )
    def sc_pass(g_hbm, src_hbm, dst_hbm, zeros_hbm, out_hbm,
                srcs_v, dsts_v, msg_v, acc_sh, gsem, ssem):
        c = lax.axis_index("c")
        s = lax.axis_index("s")
        r0 = s * ROWS_PER_TILE
        wid = s * NC + c

        isem = gsem
        idesc = pltpu.async_copy(src_hbm.at[wid], srcs_v, isem)
        idesc2 = pltpu.async_copy(dst_hbm.at[wid], dsts_v, isem)

        @pl.when(c == 0)
        def _():
            pltpu.sync_copy(g_hbm.at[pl.ds(r0, ROWS_PER_TILE)],
                            acc_sh.at[pl.ds(r0, ROWS_PER_TILE)])

        @pl.when(c != 0)
        def _():
            pltpu.sync_copy(zeros_hbm.at[pl.ds(r0, ROWS_PER_TILE)],
                            acc_sh.at[pl.ds(r0, ROWS_PER_TILE)])

        idesc.wait()
        idesc2.wait()
        plsc.subcore_barrier()

        def issue_gathers(grp, pbuf):
            base = grp * NBUF
            return [
                pltpu.async_copy(g_hbm.at[srcs_v.at[base + b]],
                                 msg_v.at[pbuf].at[b], gsem)
                for b in range(NBUF)
            ]

        def issue_scatters(grp, pbuf):
            base = grp * NBUF
            return [
                pltpu.async_copy(msg_v.at[pbuf].at[b],
                                 acc_sh.at[dsts_v.at[base + b]],
                                 ssem, add=True)
                for b in range(NBUF)
            ]

        for d in issue_gathers(0, 0):
            d.wait()

        @pl.loop(0, NGROUP - 1)
        def _(grp):
            pbuf = lax.rem(grp, 2)
            sds = issue_scatters(grp, pbuf)
            gds = issue_gathers(grp + 1, 1 - pbuf)
            for d in sds:
                d.wait()
            for d in gds:
                d.wait()

        for d in issue_scatters(NGROUP - 1, lax.rem(NGROUP - 1, 2)):
            d.wait()

        plsc.subcore_barrier()
        pltpu.sync_copy(acc_sh.at[pl.ds(r0, ROWS_PER_TILE)],
                        out_hbm.at[c].at[pl.ds(r0, ROWS_PER_TILE)])

    return sc_pass


@functools.lru_cache(maxsize=None)
def _make_deg_pass():
    @functools.partial(
        pl.kernel,
        out_type=jax.ShapeDtypeStruct((NC, NP, DEGW), jnp.float32),
        mesh=_mesh(),
        compiler_params=pltpu.CompilerParams(use_tc_tiling_on_sc=False),
        scratch_types=[
            pltpu.VMEM((NCHUNK, CHUNK), jnp.int32),
            pltpu.VMEM((CHUNK, DEGW), jnp.float32),
            pltpu.VMEM_SHARED((NP, DEGW), jnp.float32),
            pltpu.SemaphoreType.DMA,
        ],
    )
    def deg_pass(dst_hbm, ones_hbm, zeros_hbm, out_hbm,
                 dsts_v, ones_v, acc_sh, ssem):
        c = lax.axis_index("c")
        s = lax.axis_index("s")
        r0 = s * ROWS_PER_TILE
        wid = s * NC + c
        pltpu.sync_copy(dst_hbm.at[wid], dsts_v)
        pltpu.sync_copy(zeros_hbm.at[pl.ds(r0, ROWS_PER_TILE)],
                        acc_sh.at[pl.ds(r0, ROWS_PER_TILE)])
        pltpu.sync_copy(ones_hbm, ones_v)
        plsc.subcore_barrier()

        DEG_NBUF = 25

        @pl.loop(0, NCHUNK // DEG_NBUF)
        def _(grp):
            base = grp * DEG_NBUF
            sds = [
                pltpu.async_copy(ones_v,
                                 acc_sh.at[dsts_v.at[base + b]],
                                 ssem, add=True)
                for b in range(DEG_NBUF)
            ]
            for d in sds:
                d.wait()

        plsc.subcore_barrier()
        pltpu.sync_copy(acc_sh.at[pl.ds(r0, ROWS_PER_TILE)],
                        out_hbm.at[c].at[pl.ds(r0, ROWS_PER_TILE)])

    return deg_pass


_R = 640
_GRID = NP // _R


def _rows(shape_tail):
    ntail = len(shape_tail)
    return pl.BlockSpec((_R,) + shape_tail, lambda i: (i,) + (0,) * ntail)


def _full(shape):
    nd = len(shape)
    return pl.BlockSpec(shape, lambda i: (0,) * nd)


def _acc_spec(w):
    return pl.BlockSpec((NC, _R, w), lambda i: (0, i, 0))


def _tc_call(body, in_specs, out_specs, out_shape):
    return pl.pallas_call(
        body,
        grid=(_GRID,),
        in_specs=in_specs,
        out_specs=out_specs,
        out_shape=out_shape,
    )


def _tc_a(degp, x, W1):
    def body(degp_ref, x_ref, w1_ref, dinv_ref, g1_ref):
        deg = degp_ref[0] + degp_ref[1] + 1.0
        dinv8 = lax.rsqrt(deg)
        dinv = dinv8[:, 0:1]
        dinv_ref[...] = dinv
        h = jnp.dot(x_ref[...], w1_ref[...], preferred_element_type=jnp.float32)
        g1_ref[...] = h * dinv

    return _tc_call(
        body,
        [_acc_spec(DEGW), _rows((128,)), _full((128, 64))],
        [_rows((1,)), _rows((64,))],
        [jax.ShapeDtypeStruct((NP, 1), jnp.float32),
         jax.ShapeDtypeStruct((NP, 64), jnp.float32)],
    )(degp, x, W1)


def _tc_b(acc1, dinv, b1):
    def body(acc_ref, dinv_ref, b_ref, out_ref):
        dinv = dinv_ref[...]
        t = (acc_ref[0] + acc_ref[1]) * dinv + b_ref[...]
        out_ref[...] = jnp.maximum(t, 0.0) * dinv

    return _tc_call(
        body,
        [_acc_spec(64), _rows((1,)), _full((1, 64))],
        _rows((64,)),
        jax.ShapeDtypeStruct((NP, 64), jnp.float32),
    )(acc1, dinv, b1.reshape(1, -1))


def _tc_c(acc2, dinv, W2, b2, W3):
    def body(acc_ref, dinv_ref, w2_ref, b2_ref, w3_ref, out_ref):
        dinv = dinv_ref[...]
        s2 = (acc_ref[0] + acc_ref[1]) * dinv
        x3 = jnp.maximum(
            jnp.dot(s2, w2_ref[...], preferred_element_type=jnp.float32)
            + b2_ref[...], 0.0)
        h3 = jnp.dot(x3, w3_ref[...], preferred_element_type=jnp.float32)
        out_ref[...] = h3 * dinv

    return _tc_call(
        body,
        [_acc_spec(64), _rows((1,)),
         _full((64, 128)), _full((1, 128)), _full((128, 64))],
        _rows((64,)),
        jax.ShapeDtypeStruct((NP, 64), jnp.float32),
    )(acc2, dinv, W2, b2.reshape(1, -1), W3)


def _tc_d(acc3, dinv, b3, W4):
    def body(acc_ref, dinv_ref, b_ref, w4_ref, out_ref):
        dinv = dinv_ref[...]
        x4 = jnp.maximum((acc_ref[0] + acc_ref[1]) * dinv
                         + b_ref[...], 0.0)
        h4 = jnp.dot(x4, w4_ref[...], preferred_element_type=jnp.float32)
        out_ref[...] = h4 * dinv

    return _tc_call(
        body,
        [_acc_spec(64), _rows((1,)), _full((1, 64)),
         _full((64, 16))],
        _rows((16,)),
        jax.ShapeDtypeStruct((NP, 16), jnp.float32),
    )(acc3, dinv, b3.reshape(1, -1), W4)


def _tc_e(acc4, dinv, b4):
    def body(acc_ref, dinv_ref, b_ref, out_ref):
        dinv = dinv_ref[...]
        x5 = jnp.maximum((acc_ref[0] + acc_ref[1]) * dinv
                         + b_ref[...], 0.0)
        out_ref[...] = x5 * dinv

    return _tc_call(
        body,
        [_acc_spec(16), _rows((1,)), _full((1, 16))],
        _rows((16,)),
        jax.ShapeDtypeStruct((NP, 16), jnp.float32),
    )(acc4, dinv, b4.reshape(1, -1))


def _tc_f(acc5, dinv, W5, b5):
    def body(acc_ref, dinv_ref, w5_ref, b5_ref, out_ref):
        dinv = dinv_ref[...]
        s5 = (acc_ref[0] + acc_ref[1]) * dinv
        z = jnp.dot(s5, w5_ref[...], preferred_element_type=jnp.float32) + b5_ref[...]
        m = jnp.max(z, axis=1, keepdims=True)
        lse = m + jnp.log(jnp.sum(jnp.exp(z - m), axis=1, keepdims=True))
        out_ref[...] = z - lse

    return _tc_call(
        body,
        [_acc_spec(16), _rows((1,)),
         _full((16, 2)), _full((1, 2))],
        _rows((2,)),
        jax.ShapeDtypeStruct((NP, 2), jnp.float32),
    )(acc5, dinv, W5, b5.reshape(1, -1))


def kernel(x, edge_index, W1, b1, W2, b2, W3, b3, W4, b4, W5, b5):
    src = edge_index[0].reshape(NW, NCHUNK, CHUNK)
    dst = edge_index[1].reshape(NW, NCHUNK, CHUNK)
    x = jnp.pad(x, ((0, NP - N), (0, 0)))
    zeros64 = jnp.zeros((NP, 64), jnp.float32)
    zeros16 = jnp.zeros((NP, 16), jnp.float32)
    zeros8 = jnp.zeros((NP, DEGW), jnp.float32)
    ones = jnp.ones((CHUNK, DEGW), jnp.float32)

    sc64 = _make_sc_pass(64)
    sc16 = _make_sc_pass(16)

    degp = _make_deg_pass()(dst, ones, zeros8)
    dinv, g1 = _tc_a(degp, x, W1)

    acc1 = sc64(g1, src, dst, zeros64)
    g2 = _tc_b(acc1, dinv, b1)

    acc2 = sc64(g2, src, dst, zeros64)
    g3 = _tc_c(acc2, dinv, W2, b2, W3)

    acc3 = sc64(g3, src, dst, zeros64)
    g4 = _tc_d(acc3, dinv, b3, W4)

    acc4 = sc16(g4, src, dst, zeros16)
    g5 = _tc_e(acc4, dinv, b4)

    acc5 = sc16(g5, src, dst, zeros16)
    return _tc_f(acc5, dinv, W5, b5)[:N]


# speedup vs baseline: 37.3829x; 1.0791x over previous
import functools

import jax
import jax.numpy as jnp
from jax import lax
from jax.experimental import pallas as pl
from jax.experimental.pallas import tpu as pltpu
from jax.experimental.pallas import tpu_sc as plsc

N = 10000
NP = 10240
E = 320000
NC = 2
NS = 16
NW = NC * NS
EDGES_PER_W = E // NW
CHUNK = 80
NCHUNK = EDGES_PER_W // CHUNK
NBUF = 5
NGROUP = NCHUNK // NBUF
ROWS_PER_TILE = NP // NS
DEGW = 8

_mesh = lambda: plsc.VectorSubcoreMesh(core_axis_name="c", subcore_axis_name="s")


@functools.lru_cache(maxsize=None)
def _make_sc_pass(w):
    @functools.partial(
        pl.kernel,
        out_type=jax.ShapeDtypeStruct((NC, NP, w), jnp.float32),
        mesh=_mesh(),
        compiler_params=pltpu.CompilerParams(use_tc_tiling_on_sc=False),
        scratch_types=[
            pltpu.VMEM((NCHUNK, CHUNK), jnp.int32),
            pltpu.VMEM((NCHUNK, CHUNK), jnp.int32),
            pltpu.VMEM((2, NBUF, CHUNK, w), jnp.float32),
            pltpu.VMEM_SHARED((NP, w), jnp.float32),
            pltpu.SemaphoreType.DMA,
            pltpu.SemaphoreType.DMA,
        ],
    )
    def sc_pass(g_hbm, src_hbm, dst_hbm, zeros_hbm, out_hbm,
                srcs_v, dsts_v, msg_v, acc_sh, gsem, ssem):
        c = lax.axis_index("c")
        s = lax.axis_index("s")
        r0 = s * ROWS_PER_TILE
        wid = s * NC + c

        isem = gsem
        idesc = pltpu.async_copy(src_hbm.at[wid], srcs_v, isem)
        idesc2 = pltpu.async_copy(dst_hbm.at[wid], dsts_v, isem)

        @pl.when(c == 0)
        def _():
            pltpu.sync_copy(g_hbm.at[pl.ds(r0, ROWS_PER_TILE)],
                            acc_sh.at[pl.ds(r0, ROWS_PER_TILE)])

        @pl.when(c != 0)
        def _():
            pltpu.sync_copy(zeros_hbm.at[pl.ds(r0, ROWS_PER_TILE)],
                            acc_sh.at[pl.ds(r0, ROWS_PER_TILE)])

        idesc.wait()
        idesc2.wait()
        plsc.subcore_barrier()

        def issue_gathers(grp, pbuf):
            base = grp * NBUF
            return [
                pltpu.async_copy(g_hbm.at[srcs_v.at[base + b]],
                                 msg_v.at[pbuf].at[b], gsem)
                for b in range(NBUF)
            ]

        def issue_scatters(grp, pbuf):
            base = grp * NBUF
            return [
                pltpu.async_copy(msg_v.at[pbuf].at[b],
                                 acc_sh.at[dsts_v.at[base + b]],
                                 ssem, add=True)
                for b in range(NBUF)
            ]

        for d in issue_gathers(0, 0):
            d.wait()

        @pl.loop(0, NGROUP - 1)
        def _(grp):
            pbuf = lax.rem(grp, 2)
            sds = issue_scatters(grp, pbuf)
            gds = issue_gathers(grp + 1, 1 - pbuf)
            for d in sds:
                d.wait()
            for d in gds:
                d.wait()

        for d in issue_scatters(NGROUP - 1, lax.rem(NGROUP - 1, 2)):
            d.wait()

        plsc.subcore_barrier()
        pltpu.sync_copy(acc_sh.at[pl.ds(r0, ROWS_PER_TILE)],
                        out_hbm.at[c].at[pl.ds(r0, ROWS_PER_TILE)])

    return sc_pass


@functools.lru_cache(maxsize=None)
def _make_deg_pass():
    @functools.partial(
        pl.kernel,
        out_type=jax.ShapeDtypeStruct((NC, NP, DEGW), jnp.float32),
        mesh=_mesh(),
        compiler_params=pltpu.CompilerParams(use_tc_tiling_on_sc=False),
        scratch_types=[
            pltpu.VMEM((NCHUNK, CHUNK), jnp.int32),
            pltpu.VMEM((CHUNK, DEGW), jnp.float32),
            pltpu.VMEM_SHARED((NP, DEGW), jnp.float32),
            pltpu.SemaphoreType.DMA,
        ],
    )
    def deg_pass(dst_hbm, ones_hbm, zeros_hbm, out_hbm,
                 dsts_v, ones_v, acc_sh, ssem):
        c = lax.axis_index("c")
        s = lax.axis_index("s")
        r0 = s * ROWS_PER_TILE
        wid = s * NC + c
        pltpu.sync_copy(dst_hbm.at[wid], dsts_v)
        pltpu.sync_copy(zeros_hbm.at[pl.ds(r0, ROWS_PER_TILE)],
                        acc_sh.at[pl.ds(r0, ROWS_PER_TILE)])
        pltpu.sync_copy(ones_hbm, ones_v)
        plsc.subcore_barrier()

        DEG_NBUF = 25

        @pl.loop(0, NCHUNK // DEG_NBUF)
        def _(grp):
            base = grp * DEG_NBUF
            sds = [
                pltpu.async_copy(ones_v,
                                 acc_sh.at[dsts_v.at[base + b]],
                                 ssem, add=True)
                for b in range(DEG_NBUF)
            ]
            for d in sds:
                d.wait()

        plsc.subcore_barrier()
        pltpu.sync_copy(acc_sh.at[pl.ds(r0, ROWS_PER_TILE)],
                        out_hbm.at[c].at[pl.ds(r0, ROWS_PER_TILE)])

    return deg_pass


_R = NP
_GRID = NP // _R


def _rows(shape_tail):
    ntail = len(shape_tail)
    return pl.BlockSpec((_R,) + shape_tail, lambda i: (i,) + (0,) * ntail)


def _full(shape):
    nd = len(shape)
    return pl.BlockSpec(shape, lambda i: (0,) * nd)


def _acc_spec(w):
    return pl.BlockSpec((NC, _R, w), lambda i: (0, i, 0))


def _tc_call(body, in_specs, out_specs, out_shape):
    return pl.pallas_call(
        body,
        grid=(_GRID,),
        in_specs=in_specs,
        out_specs=out_specs,
        out_shape=out_shape,
    )


def _tc_a(degp, x, W1):
    def body(degp_ref, x_ref, w1_ref, dinv_ref, g1_ref):
        deg = degp_ref[0] + degp_ref[1] + 1.0
        dinv8 = lax.rsqrt(deg)
        dinv = dinv8[:, 0:1]
        dinv_ref[...] = dinv
        h = jnp.dot(x_ref[...], w1_ref[...], preferred_element_type=jnp.float32)
        g1_ref[0:N, :] = h * dinv[0:N]
        g1_ref[N:NP, :] = jnp.zeros((NP - N, 64), jnp.float32)

    return _tc_call(
        body,
        [_acc_spec(DEGW), pl.BlockSpec((N, 128), lambda i: (0, 0)),
         _full((128, 64))],
        [_rows((1,)), _rows((64,))],
        [jax.ShapeDtypeStruct((NP, 1), jnp.float32),
         jax.ShapeDtypeStruct((NP, 64), jnp.float32)],
    )(degp, x, W1)


def _tc_b(acc1, dinv, b1):
    def body(acc_ref, dinv_ref, b_ref, out_ref):
        dinv = dinv_ref[...]
        t = (acc_ref[0] + acc_ref[1]) * dinv + b_ref[...]
        out_ref[...] = jnp.maximum(t, 0.0) * dinv

    return _tc_call(
        body,
        [_acc_spec(64), _rows((1,)), _full((1, 64))],
        _rows((64,)),
        jax.ShapeDtypeStruct((NP, 64), jnp.float32),
    )(acc1, dinv, b1.reshape(1, -1))


def _tc_c(acc2, dinv, W2, b2, W3):
    def body(acc_ref, dinv_ref, w2_ref, b2_ref, w3_ref, out_ref):
        dinv = dinv_ref[...]
        s2 = (acc_ref[0] + acc_ref[1]) * dinv
        x3 = jnp.maximum(
            jnp.dot(s2, w2_ref[...], preferred_element_type=jnp.float32)
            + b2_ref[...], 0.0)
        h3 = jnp.dot(x3, w3_ref[...], preferred_element_type=jnp.float32)
        out_ref[...] = h3 * dinv

    return _tc_call(
        body,
        [_acc_spec(64), _rows((1,)),
         _full((64, 128)), _full((1, 128)), _full((128, 64))],
        _rows((64,)),
        jax.ShapeDtypeStruct((NP, 64), jnp.float32),
    )(acc2, dinv, W2, b2.reshape(1, -1), W3)


def _tc_d(acc3, dinv, b3, W4):
    def body(acc_ref, dinv_ref, b_ref, w4_ref, out_ref):
        dinv = dinv_ref[...]
        x4 = jnp.maximum((acc_ref[0] + acc_ref[1]) * dinv
                         + b_ref[...], 0.0)
        h4 = jnp.dot(x4, w4_ref[...], preferred_element_type=jnp.float32)
        out_ref[...] = h4 * dinv

    return _tc_call(
        body,
        [_acc_spec(64), _rows((1,)), _full((1, 64)),
         _full((64, 16))],
        _rows((16,)),
        jax.ShapeDtypeStruct((NP, 16), jnp.float32),
    )(acc3, dinv, b3.reshape(1, -1), W4)


def _tc_e(acc4, dinv, b4):
    def body(acc_ref, dinv_ref, b_ref, out_ref):
        dinv = dinv_ref[...]
        x5 = jnp.maximum((acc_ref[0] + acc_ref[1]) * dinv
                         + b_ref[...], 0.0)
        out_ref[...] = x5 * dinv

    return _tc_call(
        body,
        [_acc_spec(16), _rows((1,)), _full((1, 16))],
        _rows((16,)),
        jax.ShapeDtypeStruct((NP, 16), jnp.float32),
    )(acc4, dinv, b4.reshape(1, -1))


def _tc_f(acc5, dinv, W5, b5):
    def body(acc_ref, dinv_ref, w5_ref, b5_ref, out_ref):
        dinv = dinv_ref[...]
        s5 = (acc_ref[0] + acc_ref[1]) * dinv
        z = jnp.dot(s5, w5_ref[...], preferred_element_type=jnp.float32) + b5_ref[...]
        m = jnp.max(z, axis=1, keepdims=True)
        lse = m + jnp.log(jnp.sum(jnp.exp(z - m), axis=1, keepdims=True))
        out_ref[...] = z - lse

    return _tc_call(
        body,
        [_acc_spec(16), _rows((1,)),
         _full((16, 2)), _full((1, 2))],
        _rows((2,)),
        jax.ShapeDtypeStruct((NP, 2), jnp.float32),
    )(acc5, dinv, W5, b5.reshape(1, -1))


def kernel(x, edge_index, W1, b1, W2, b2, W3, b3, W4, b4, W5, b5):
    src = edge_index[0].reshape(NW, NCHUNK, CHUNK)
    dst = edge_index[1].reshape(NW, NCHUNK, CHUNK)
    zeros64 = jnp.zeros((NP, 64), jnp.float32)
    zeros16 = jnp.zeros((NP, 16), jnp.float32)
    zeros8 = jnp.zeros((NP, DEGW), jnp.float32)
    ones = jnp.ones((CHUNK, DEGW), jnp.float32)

    sc64 = _make_sc_pass(64)
    sc16 = _make_sc_pass(16)

    degp = _make_deg_pass()(dst, ones, zeros8)
    dinv, g1 = _tc_a(degp, x, W1)

    acc1 = sc64(g1, src, dst, zeros64)
    g2 = _tc_b(acc1, dinv, b1)

    acc2 = sc64(g2, src, dst, zeros64)
    g3 = _tc_c(acc2, dinv, W2, b2, W3)

    acc3 = sc64(g3, src, dst, zeros64)
    g4 = _tc_d(acc3, dinv, b3, W4)

    acc4 = sc16(g4, src, dst, zeros16)
    g5 = _tc_e(acc4, dinv, b4)

    acc5 = sc16(g5, src, dst, zeros16)
    return _tc_f(acc5, dinv, W5, b5)[:N]


# speedup vs baseline: 49.3507x; 1.3201x over previous
import functools

import jax
import jax.numpy as jnp
from jax import lax
from jax.experimental import pallas as pl
from jax.experimental.pallas import tpu as pltpu
from jax.experimental.pallas import tpu_sc as plsc

N = 10000
NP = 10240
E = 320000
NC = 2
NS = 16
NW = NC * NS
EDGES_PER_W = E // NW
CHUNK = 80
NCHUNK = EDGES_PER_W // CHUNK
NBUF = 5
NGROUP = NCHUNK // NBUF
ROWS_PER_TILE = NP // NS
DEGW = 8

_mesh = lambda: plsc.VectorSubcoreMesh(core_axis_name="c", subcore_axis_name="s")


@functools.lru_cache(maxsize=None)
def _make_sc_pass(w, stage_g=False):
    scratch = [
        pltpu.VMEM((NCHUNK, CHUNK), jnp.int32),
        pltpu.VMEM((NCHUNK, CHUNK), jnp.int32),
        pltpu.VMEM((2, NBUF, CHUNK, w), jnp.float32),
        pltpu.VMEM_SHARED((NP, w), jnp.float32),
        pltpu.SemaphoreType.DMA,
        pltpu.SemaphoreType.DMA,
    ]
    if stage_g:
        scratch.append(pltpu.VMEM_SHARED((NP, w), jnp.float32))

    @functools.partial(
        pl.kernel,
        out_type=jax.ShapeDtypeStruct((NC, NP, w), jnp.float32),
        mesh=_mesh(),
        compiler_params=pltpu.CompilerParams(use_tc_tiling_on_sc=False),
        scratch_types=scratch,
    )
    def sc_pass(g_hbm, ei_hbm, zeros_hbm, out_hbm,
                srcs_v, dsts_v, msg_v, acc_sh, gsem, ssem, *maybe_gsh):
        c = lax.axis_index("c")
        s = lax.axis_index("s")
        r0 = s * ROWS_PER_TILE
        wid = s * NC + c

        isem = gsem
        idesc = pltpu.async_copy(ei_hbm.at[0].at[wid], srcs_v, isem)
        idesc2 = pltpu.async_copy(ei_hbm.at[1].at[wid], dsts_v, isem)

        @pl.when(c == 0)
        def _():
            pltpu.sync_copy(g_hbm.at[pl.ds(r0, ROWS_PER_TILE)],
                            acc_sh.at[pl.ds(r0, ROWS_PER_TILE)])

        @pl.when(c != 0)
        def _():
            pltpu.sync_copy(zeros_hbm.at[pl.ds(r0, ROWS_PER_TILE)],
                            acc_sh.at[pl.ds(r0, ROWS_PER_TILE)])

        if stage_g:
            g_sh = maybe_gsh[0]
            pltpu.sync_copy(g_hbm.at[pl.ds(r0, ROWS_PER_TILE)],
                            g_sh.at[pl.ds(r0, ROWS_PER_TILE)])
            g_src = g_sh
        else:
            g_src = g_hbm

        idesc.wait()
        idesc2.wait()
        plsc.subcore_barrier()

        def issue_gathers(grp, pbuf):
            base = grp * NBUF
            return [
                pltpu.async_copy(g_src.at[srcs_v.at[base + b]],
                                 msg_v.at[pbuf].at[b], gsem)
                for b in range(NBUF)
            ]

        def issue_scatters(grp, pbuf):
            base = grp * NBUF
            return [
                pltpu.async_copy(msg_v.at[pbuf].at[b],
                                 acc_sh.at[dsts_v.at[base + b]],
                                 ssem, add=True)
                for b in range(NBUF)
            ]

        for d in issue_gathers(0, 0):
            d.wait()

        @pl.loop(0, NGROUP - 1)
        def _(grp):
            pbuf = lax.rem(grp, 2)
            sds = issue_scatters(grp, pbuf)
            gds = issue_gathers(grp + 1, 1 - pbuf)
            for d in sds:
                d.wait()
            for d in gds:
                d.wait()

        for d in issue_scatters(NGROUP - 1, lax.rem(NGROUP - 1, 2)):
            d.wait()

        plsc.subcore_barrier()
        pltpu.sync_copy(acc_sh.at[pl.ds(r0, ROWS_PER_TILE)],
                        out_hbm.at[c].at[pl.ds(r0, ROWS_PER_TILE)])

    return sc_pass


@functools.lru_cache(maxsize=None)
def _make_deg_pass():
    @functools.partial(
        pl.kernel,
        out_type=jax.ShapeDtypeStruct((NC, NP, DEGW), jnp.float32),
        mesh=_mesh(),
        compiler_params=pltpu.CompilerParams(use_tc_tiling_on_sc=False),
        scratch_types=[
            pltpu.VMEM((NCHUNK, CHUNK), jnp.int32),
            pltpu.VMEM((CHUNK, DEGW), jnp.float32),
            pltpu.VMEM_SHARED((NP, DEGW), jnp.float32),
            pltpu.SemaphoreType.DMA,
        ],
    )
    def deg_pass(ei_hbm, ones_hbm, zeros_hbm, out_hbm,
                 dsts_v, ones_v, acc_sh, ssem):
        c = lax.axis_index("c")
        s = lax.axis_index("s")
        r0 = s * ROWS_PER_TILE
        wid = s * NC + c
        pltpu.sync_copy(ei_hbm.at[1].at[wid], dsts_v)
        pltpu.sync_copy(zeros_hbm.at[pl.ds(r0, ROWS_PER_TILE)],
                        acc_sh.at[pl.ds(r0, ROWS_PER_TILE)])
        pltpu.sync_copy(ones_hbm, ones_v)
        plsc.subcore_barrier()

        DEG_NBUF = 25

        @pl.loop(0, NCHUNK // DEG_NBUF)
        def _(grp):
            base = grp * DEG_NBUF
            sds = [
                pltpu.async_copy(ones_v,
                                 acc_sh.at[dsts_v.at[base + b]],
                                 ssem, add=True)
                for b in range(DEG_NBUF)
            ]
            for d in sds:
                d.wait()

        plsc.subcore_barrier()
        pltpu.sync_copy(acc_sh.at[pl.ds(r0, ROWS_PER_TILE)],
                        out_hbm.at[c].at[pl.ds(r0, ROWS_PER_TILE)])

    return deg_pass


M2 = NP // 2
M8 = NP // 8


def _full(shape):
    nd = len(shape)
    return pl.BlockSpec(shape, lambda: (0,) * nd)


def _tc_call(body, in_specs, out_specs, out_shape):
    return pl.pallas_call(
        body,
        grid=(),
        in_specs=in_specs,
        out_specs=out_specs,
        out_shape=out_shape,
    )


def _tc_h1(xv, W1L, W1R):
    def body(x_ref, wl_ref, wr_ref, h_ref):
        h_ref[...] = (jnp.dot(x_ref[:, 0, :], wl_ref[...],
                              preferred_element_type=jnp.float32)
                      + jnp.dot(x_ref[:, 1, :], wr_ref[...],
                                preferred_element_type=jnp.float32))

    return _tc_call(
        body,
        [_full((N // 2, 2, 128)), _full((128, 128)), _full((128, 128))],
        _full((N // 2, 128)),
        jax.ShapeDtypeStruct((N // 2, 128), jnp.float32),
    )(xv, W1L, W1R)


def _tc_a(degp16, degp64, h):
    def body(d16_ref, d64_ref, h_ref, dinvE_ref, dinvE8_ref, g1p_ref):
        d16 = d16_ref[0] + d16_ref[1] + 1.0
        dv16 = lax.rsqrt(d16)
        dinvE = jnp.concatenate(
            [jnp.broadcast_to(dv16[:, 0:1], (M2, 64)),
             jnp.broadcast_to(dv16[:, 8:9], (M2, 64))], axis=1)
        dinvE_ref[...] = dinvE
        d64 = d64_ref[0] + d64_ref[1] + 1.0
        dv64 = lax.rsqrt(d64)
        dinvE8_ref[...] = jnp.concatenate(
            [jnp.broadcast_to(dv64[:, 8 * a:8 * a + 1], (M8, 16))
             for a in range(8)], axis=1)
        g1p_ref[0:N // 2, :] = h_ref[...] * dinvE[0:N // 2]
        g1p_ref[N // 2:M2, :] = jnp.zeros((M2 - N // 2, 128), jnp.float32)

    return _tc_call(
        body,
        [_full((NC, M2, 16)), _full((NC, M8, 64)), _full((N // 2, 128))],
        [_full((M2, 128)), _full((M8, 128)), _full((M2, 128))],
        [jax.ShapeDtypeStruct((M2, 128), jnp.float32),
         jax.ShapeDtypeStruct((M8, 128), jnp.float32),
         jax.ShapeDtypeStruct((M2, 128), jnp.float32)],
    )(degp16, degp64, h)


def _tc_b(accp, dinvE, b1t):
    def body(acc_ref, dinvE_ref, b_ref, out_ref):
        dinvE = dinvE_ref[...]
        t = (acc_ref[0] + acc_ref[1]) * dinvE + b_ref[...]
        out_ref[...] = jnp.maximum(t, 0.0) * dinvE

    return _tc_call(
        body,
        [_full((NC, M2, 128)), _full((M2, 128)), _full((1, 128))],
        _full((M2, 128)),
        jax.ShapeDtypeStruct((M2, 128), jnp.float32),
    )(accp, dinvE, b1t)


def _tc_c(accp, dinvE, W2bd, b2t, W3bd):
    def body(acc_ref, dinvE_ref, w2_ref, b2_ref, w3_ref, out_ref):
        dinvE = dinvE_ref[...]
        s2p = (acc_ref[0] + acc_ref[1]) * dinvE
        x3p = jnp.maximum(
            jnp.dot(s2p, w2_ref[...], preferred_element_type=jnp.float32)
            + b2_ref[...], 0.0)
        h3p = jnp.dot(x3p, w3_ref[...], preferred_element_type=jnp.float32)
        out_ref[...] = h3p * dinvE

    return _tc_call(
        body,
        [_full((NC, M2, 128)), _full((M2, 128)),
         _full((128, 256)), _full((1, 256)), _full((256, 128))],
        _full((M2, 128)),
        jax.ShapeDtypeStruct((M2, 128), jnp.float32),
    )(accp, dinvE, W2bd, b2t, W3bd)


def _tc_d(accp, dinvE, b3t, K, dinvE8):
    def body(acc_ref, dinvE_ref, b_ref, k_ref, dinvE8_ref, out_ref):
        dinvE = dinvE_ref[...]
        x4p = jnp.maximum((acc_ref[0] + acc_ref[1]) * dinvE + b_ref[...],
                          0.0)
        x4v = x4p.reshape(M8, 4, 128)
        h = jnp.dot(x4v[:, 0, :], k_ref[0], preferred_element_type=jnp.float32)
        for a in range(1, 4):
            h = h + jnp.dot(x4v[:, a, :], k_ref[a],
                            preferred_element_type=jnp.float32)
        out_ref[...] = h * dinvE8_ref[...]

    return _tc_call(
        body,
        [_full((NC, M2, 128)), _full((M2, 128)), _full((1, 128)),
         _full((4, 128, 128)), _full((M8, 128))],
        _full((M8, 128)),
        jax.ShapeDtypeStruct((M8, 128), jnp.float32),
    )(accp, dinvE, b3t, K, dinvE8)


def _tc_e(accp8, dinvE8, b4t8):
    def body(acc_ref, dinvE8_ref, b_ref, out_ref):
        dinvE8 = dinvE8_ref[...]
        t = (acc_ref[0] + acc_ref[1]) * dinvE8 + b_ref[...]
        out_ref[...] = jnp.maximum(t, 0.0) * dinvE8

    return _tc_call(
        body,
        [_full((NC, M8, 128)), _full((M8, 128)), _full((1, 128))],
        _full((M8, 128)),
        jax.ShapeDtypeStruct((M8, 128), jnp.float32),
    )(accp8, dinvE8, b4t8)


def _tc_f(accp8, dinvE8, W5bd8, b5t8, P):
    def body(acc_ref, dinvE8_ref, w5_ref, b5_ref, p_ref, out_ref):
        s5 = (acc_ref[0] + acc_ref[1]) * dinvE8_ref[...]
        z = jnp.dot(s5, w5_ref[...],
                    preferred_element_type=jnp.float32) + b5_ref[...]
        zs = jnp.dot(z, p_ref[...], preferred_element_type=jnp.float32)
        m = jnp.maximum(z, zs)
        lse = m + jnp.log(jnp.exp(z - m) + jnp.exp(zs - m))
        out_ref[...] = z - lse

    return _tc_call(
        body,
        [_full((NC, M8, 128)), _full((M8, 128)),
         _full((128, 16)), _full((1, 16)), _full((16, 16))],
        _full((M8, 16)),
        jax.ShapeDtypeStruct((M8, 16), jnp.float32),
    )(accp8, dinvE8, W5bd8, b5t8, P)


def kernel(x, edge_index, W1, b1, W2, b2, W3, b3, W4, b4, W5, b5):
    ei = edge_index.reshape(2, NW, NCHUNK, CHUNK)
    zeros64 = jnp.zeros((NP, 64), jnp.float32)
    zeros16 = jnp.zeros((NP, 16), jnp.float32)
    zeros8 = jnp.zeros((NP, DEGW), jnp.float32)
    ones = jnp.ones((CHUNK, DEGW), jnp.float32)

    f32 = jnp.float32
    zW1 = jnp.zeros_like(W1)
    W1L = jnp.concatenate([W1, zW1], axis=1)
    W1R = jnp.concatenate([zW1, W1], axis=1)
    eye2 = jnp.eye(2, dtype=f32)
    W2bd = jnp.kron(eye2, W2)
    W3bd = jnp.kron(eye2, W3)
    Kbase = jnp.kron(eye2, W4)
    K = jnp.stack([jnp.pad(Kbase, ((0, 0), (32 * a, 96 - 32 * a)))
                   for a in range(4)])
    W5bd8 = jnp.kron(jnp.eye(8, dtype=f32), W5)
    P = jnp.eye(16, dtype=f32)[jnp.arange(16) ^ 1]
    b1t = jnp.tile(b1, 2)[None]
    b2t = jnp.tile(b2, 2)[None]
    b3t = jnp.tile(b3, 2)[None]
    b4t8 = jnp.tile(b4, 8)[None]
    b5t8 = jnp.tile(b5, 8)[None]

    sc64 = _make_sc_pass(64)
    sc16 = _make_sc_pass(16)

    h1 = _tc_h1(x.reshape(N // 2, 2, 128), W1L, W1R)
    degp = _make_deg_pass()(ei, ones, zeros8)
    dinvE, dinvE8, g1p = _tc_a(degp.reshape(NC, M2, 16),
                               degp.reshape(NC, M8, 64), h1)

    acc1 = sc64(g1p.reshape(NP, 64), ei, zeros64)
    g2p = _tc_b(acc1.reshape(NC, M2, 128), dinvE, b1t)

    acc2 = sc64(g2p.reshape(NP, 64), ei, zeros64)
    g3p = _tc_c(acc2.reshape(NC, M2, 128), dinvE, W2bd, b2t, W3bd)

    acc3 = sc64(g3p.reshape(NP, 64), ei, zeros64)
    g4p8 = _tc_d(acc3.reshape(NC, M2, 128), dinvE, b3t, K, dinvE8)

    acc4 = sc16(g4p8.reshape(NP, 16), ei, zeros16)
    g5p8 = _tc_e(acc4.reshape(NC, M8, 128), dinvE8, b4t8)

    acc5 = sc16(g5p8.reshape(NP, 16), ei, zeros16)
    z = _tc_f(acc5.reshape(NC, M8, 128), dinvE8, W5bd8, b5t8, P)
    return z[:N // 8].reshape(N, 2)


# speedup vs baseline: 52.2616x; 1.0590x over previous
import functools

import jax
import jax.numpy as jnp
from jax import lax
from jax.experimental import pallas as pl
from jax.experimental.pallas import tpu as pltpu
from jax.experimental.pallas import tpu_sc as plsc

N = 10000
NP = 10240
E = 320000
NC = 2
NS = 16
NW = NC * NS
EDGES_PER_W = E // NW
CHUNK = 80
NCHUNK = EDGES_PER_W // CHUNK
NBUF = 5
NGROUP = NCHUNK // NBUF
ROWS_PER_TILE = NP // NS
DEGW = 8

_mesh = lambda: plsc.VectorSubcoreMesh(core_axis_name="c", subcore_axis_name="s")


@functools.lru_cache(maxsize=None)
def _make_sc_pass(w):
    nbuf = 25 if w <= 16 else NBUF
    ngroup = NCHUNK // nbuf

    @functools.partial(
        pl.kernel,
        out_type=jax.ShapeDtypeStruct((NC, NP, w), jnp.float32),
        mesh=_mesh(),
        compiler_params=pltpu.CompilerParams(use_tc_tiling_on_sc=False),
        scratch_types=[
            pltpu.VMEM((NCHUNK, CHUNK), jnp.int32),
            pltpu.VMEM((NCHUNK, CHUNK), jnp.int32),
            pltpu.VMEM((2, nbuf, CHUNK, w), jnp.float32),
            pltpu.VMEM_SHARED((NP, w), jnp.float32),
            pltpu.SemaphoreType.DMA,
            pltpu.SemaphoreType.DMA,
        ],
    )
    def sc_pass(g_hbm, ei_hbm, zeros_hbm, out_hbm,
                srcs_v, dsts_v, msg_v, acc_sh, gsem, ssem):
        c = lax.axis_index("c")
        s = lax.axis_index("s")
        r0 = s * ROWS_PER_TILE
        wid = s * NC + c

        idesc = pltpu.async_copy(ei_hbm.at[0].at[wid], srcs_v, gsem)
        idesc2 = pltpu.async_copy(ei_hbm.at[1].at[wid], dsts_v, gsem)

        @pl.when(c == 0)
        def _():
            pltpu.sync_copy(g_hbm.at[pl.ds(r0, ROWS_PER_TILE)],
                            acc_sh.at[pl.ds(r0, ROWS_PER_TILE)])

        @pl.when(c != 0)
        def _():
            pltpu.sync_copy(zeros_hbm.at[pl.ds(r0, ROWS_PER_TILE)],
                            acc_sh.at[pl.ds(r0, ROWS_PER_TILE)])

        idesc.wait()
        idesc2.wait()
        plsc.subcore_barrier()

        def issue_gathers(grp, pbuf):
            base = grp * nbuf
            return [
                pltpu.async_copy(g_hbm.at[srcs_v.at[base + b]],
                                 msg_v.at[pbuf].at[b], gsem)
                for b in range(nbuf)
            ]

        def issue_scatters(grp, pbuf):
            base = grp * nbuf
            return [
                pltpu.async_copy(msg_v.at[pbuf].at[b],
                                 acc_sh.at[dsts_v.at[base + b]],
                                 ssem, add=True)
                for b in range(nbuf)
            ]

        for d in issue_gathers(0, 0):
            d.wait()

        @pl.loop(0, ngroup - 1)
        def _(grp):
            pbuf = lax.rem(grp, 2)
            sds = issue_scatters(grp, pbuf)
            gds = issue_gathers(grp + 1, 1 - pbuf)
            for d in sds:
                d.wait()
            for d in gds:
                d.wait()

        for d in issue_scatters(ngroup - 1, lax.rem(ngroup - 1, 2)):
            d.wait()

        plsc.subcore_barrier()
        pltpu.sync_copy(acc_sh.at[pl.ds(r0, ROWS_PER_TILE)],
                        out_hbm.at[c].at[pl.ds(r0, ROWS_PER_TILE)])

    return sc_pass


@functools.lru_cache(maxsize=None)
def _make_deg_pass():
    @functools.partial(
        pl.kernel,
        out_type=jax.ShapeDtypeStruct((NC, NP, DEGW), jnp.float32),
        mesh=_mesh(),
        compiler_params=pltpu.CompilerParams(use_tc_tiling_on_sc=False),
        scratch_types=[
            pltpu.VMEM((NCHUNK, CHUNK), jnp.int32),
            pltpu.VMEM((CHUNK, DEGW), jnp.float32),
            pltpu.VMEM_SHARED((NP, DEGW), jnp.float32),
            pltpu.SemaphoreType.DMA,
        ],
    )
    def deg_pass(ei_hbm, ones_hbm, zeros_hbm, out_hbm,
                 dsts_v, ones_v, acc_sh, ssem):
        c = lax.axis_index("c")
        s = lax.axis_index("s")
        r0 = s * ROWS_PER_TILE
        wid = s * NC + c
        pltpu.sync_copy(ei_hbm.at[1].at[wid], dsts_v)
        pltpu.sync_copy(zeros_hbm.at[pl.ds(r0, ROWS_PER_TILE)],
                        acc_sh.at[pl.ds(r0, ROWS_PER_TILE)])
        pltpu.sync_copy(ones_hbm, ones_v)
        plsc.subcore_barrier()

        DEG_NBUF = 25

        @pl.loop(0, NCHUNK // DEG_NBUF)
        def _(grp):
            base = grp * DEG_NBUF
            sds = [
                pltpu.async_copy(ones_v,
                                 acc_sh.at[dsts_v.at[base + b]],
                                 ssem, add=True)
                for b in range(DEG_NBUF)
            ]
            for d in sds:
                d.wait()

        plsc.subcore_barrier()
        pltpu.sync_copy(acc_sh.at[pl.ds(r0, ROWS_PER_TILE)],
                        out_hbm.at[c].at[pl.ds(r0, ROWS_PER_TILE)])

    return deg_pass


M2 = NP // 2
M8 = NP // 8


def _full(shape):
    nd = len(shape)
    return pl.BlockSpec(shape, lambda: (0,) * nd)


def _tc_call(body, in_specs, out_specs, out_shape):
    return pl.pallas_call(
        body,
        grid=(),
        in_specs=in_specs,
        out_specs=out_specs,
        out_shape=out_shape,
    )


def _tc_h1(xv, W1L, W1R):
    def body(x_ref, wl_ref, wr_ref, h_ref):
        h_ref[...] = (jnp.dot(x_ref[:, 0, :], wl_ref[...],
                              preferred_element_type=jnp.float32)
                      + jnp.dot(x_ref[:, 1, :], wr_ref[...],
                                preferred_element_type=jnp.float32))

    return _tc_call(
        body,
        [_full((N // 2, 2, 128)), _full((128, 128)), _full((128, 128))],
        _full((N // 2, 128)),
        jax.ShapeDtypeStruct((N // 2, 128), jnp.float32),
    )(xv, W1L, W1R)


def _tc_a(degp16, degp64, h):
    def body(d16_ref, d64_ref, h_ref, dinvE_ref, dinvE8_ref, g1p_ref):
        d16 = d16_ref[0] + d16_ref[1] + 1.0
        dv16 = lax.rsqrt(d16)
        dinvE = jnp.concatenate(
            [jnp.broadcast_to(dv16[:, 0:1], (M2, 64)),
             jnp.broadcast_to(dv16[:, 8:9], (M2, 64))], axis=1)
        dinvE_ref[...] = dinvE
        d64 = d64_ref[0] + d64_ref[1] + 1.0
        dv64 = lax.rsqrt(d64)
        dinvE8_ref[...] = jnp.concatenate(
            [jnp.broadcast_to(dv64[:, 8 * a:8 * a + 1], (M8, 16))
             for a in range(8)], axis=1)
        g1p_ref[0:N // 2, :] = h_ref[...] * dinvE[0:N // 2]
        g1p_ref[N // 2:M2, :] = jnp.zeros((M2 - N // 2, 128), jnp.float32)

    return _tc_call(
        body,
        [_full((NC, M2, 16)), _full((NC, M8, 64)), _full((N // 2, 128))],
        [_full((M2, 128)), _full((M8, 128)), _full((M2, 128))],
        [jax.ShapeDtypeStruct((M2, 128), jnp.float32),
         jax.ShapeDtypeStruct((M8, 128), jnp.float32),
         jax.ShapeDtypeStruct((M2, 128), jnp.float32)],
    )(degp16, degp64, h)


def _tc_b(accp, dinvE, b1t):
    def body(acc_ref, dinvE_ref, b_ref, out_ref):
        dinvE = dinvE_ref[...]
        t = (acc_ref[0] + acc_ref[1]) * dinvE + b_ref[...]
        out_ref[...] = jnp.maximum(t, 0.0) * dinvE

    return _tc_call(
        body,
        [_full((NC, M2, 128)), _full((M2, 128)), _full((1, 128))],
        _full((M2, 128)),
        jax.ShapeDtypeStruct((M2, 128), jnp.float32),
    )(accp, dinvE, b1t)


def _tc_c(accp, dinvE, W2bd, b2t, W3bd):
    def body(acc_ref, dinvE_ref, w2_ref, b2_ref, w3_ref, out_ref):
        dinvE = dinvE_ref[...]
        s2p = (acc_ref[0] + acc_ref[1]) * dinvE
        x3p = jnp.maximum(
            jnp.dot(s2p, w2_ref[...], preferred_element_type=jnp.float32)
            + b2_ref[...], 0.0)
        h3p = jnp.dot(x3p, w3_ref[...], preferred_element_type=jnp.float32)
        out_ref[...] = h3p * dinvE

    return _tc_call(
        body,
        [_full((NC, M2, 128)), _full((M2, 128)),
         _full((128, 256)), _full((1, 256)), _full((256, 128))],
        _full((M2, 128)),
        jax.ShapeDtypeStruct((M2, 128), jnp.float32),
    )(accp, dinvE, W2bd, b2t, W3bd)


def _tc_d(accp, dinvE, b3t, K, dinvE8):
    def body(acc_ref, dinvE_ref, b_ref, k_ref, dinvE8_ref, out_ref):
        dinvE = dinvE_ref[...]
        x4p = jnp.maximum((acc_ref[0] + acc_ref[1]) * dinvE + b_ref[...],
                          0.0)
        x4v = x4p.reshape(M8, 4, 128)
        h = jnp.dot(x4v[:, 0, :], k_ref[0], preferred_element_type=jnp.float32)
        for a in range(1, 4):
            h = h + jnp.dot(x4v[:, a, :], k_ref[a],
                            preferred_element_type=jnp.float32)
        out_ref[...] = h * dinvE8_ref[...]

    return _tc_call(
        body,
        [_full((NC, M2, 128)), _full((M2, 128)), _full((1, 128)),
         _full((4, 128, 128)), _full((M8, 128))],
        _full((M8, 128)),
        jax.ShapeDtypeStruct((M8, 128), jnp.float32),
    )(accp, dinvE, b3t, K, dinvE8)


def _tc_e(accp8, dinvE8, b4t8):
    def body(acc_ref, dinvE8_ref, b_ref, out_ref):
        dinvE8 = dinvE8_ref[...]
        t = (acc_ref[0] + acc_ref[1]) * dinvE8 + b_ref[...]
        out_ref[...] = jnp.maximum(t, 0.0) * dinvE8

    return _tc_call(
        body,
        [_full((NC, M8, 128)), _full((M8, 128)), _full((1, 128))],
        _full((M8, 128)),
        jax.ShapeDtypeStruct((M8, 128), jnp.float32),
    )(accp8, dinvE8, b4t8)


def _tc_f(accp8, dinvE8, W5bd8, b5t8, P):
    def body(acc_ref, dinvE8_ref, w5_ref, b5_ref, p_ref, out_ref):
        s5 = (acc_ref[0] + acc_ref[1]) * dinvE8_ref[...]
        z = jnp.dot(s5, w5_ref[...],
                    preferred_element_type=jnp.float32) + b5_ref[...]
        zs = jnp.dot(z, p_ref[...], preferred_element_type=jnp.float32)
        m = jnp.maximum(z, zs)
        lse = m + jnp.log(jnp.exp(z - m) + jnp.exp(zs - m))
        out_ref[...] = z - lse

    return _tc_call(
        body,
        [_full((NC, M8, 128)), _full((M8, 128)),
         _full((128, 16)), _full((1, 16)), _full((16, 16))],
        _full((M8, 16)),
        jax.ShapeDtypeStruct((M8, 16), jnp.float32),
    )(accp8, dinvE8, W5bd8, b5t8, P)


def kernel(x, edge_index, W1, b1, W2, b2, W3, b3, W4, b4, W5, b5):
    ei = edge_index.reshape(2, NW, NCHUNK, CHUNK)
    zeros64 = jnp.zeros((NP, 64), jnp.float32)
    zeros16 = jnp.zeros((NP, 16), jnp.float32)
    zeros8 = jnp.zeros((NP, DEGW), jnp.float32)
    ones = jnp.ones((CHUNK, DEGW), jnp.float32)

    f32 = jnp.float32
    zW1 = jnp.zeros_like(W1)
    W1L = jnp.concatenate([W1, zW1], axis=1)
    W1R = jnp.concatenate([zW1, W1], axis=1)
    eye2 = jnp.eye(2, dtype=f32)
    W2bd = jnp.kron(eye2, W2)
    W3bd = jnp.kron(eye2, W3)
    Kbase = jnp.kron(eye2, W4)
    K = jnp.stack([jnp.pad(Kbase, ((0, 0), (32 * a, 96 - 32 * a)))
                   for a in range(4)])
    W5bd8 = jnp.kron(jnp.eye(8, dtype=f32), W5)
    P = jnp.eye(16, dtype=f32)[jnp.arange(16) ^ 1]
    b1t = jnp.tile(b1, 2)[None]
    b2t = jnp.tile(b2, 2)[None]
    b3t = jnp.tile(b3, 2)[None]
    b4t8 = jnp.tile(b4, 8)[None]
    b5t8 = jnp.tile(b5, 8)[None]

    sc64 = _make_sc_pass(64)
    sc16 = _make_sc_pass(16)

    h1 = _tc_h1(x.reshape(N // 2, 2, 128), W1L, W1R)
    degp = _make_deg_pass()(ei, ones, zeros8)
    dinvE, dinvE8, g1p = _tc_a(degp.reshape(NC, M2, 16),
                               degp.reshape(NC, M8, 64), h1)

    acc1 = sc64(g1p.reshape(NP, 64), ei, zeros64)
    g2p = _tc_b(acc1.reshape(NC, M2, 128), dinvE, b1t)

    acc2 = sc64(g2p.reshape(NP, 64), ei, zeros64)
    g3p = _tc_c(acc2.reshape(NC, M2, 128), dinvE, W2bd, b2t, W3bd)

    acc3 = sc64(g3p.reshape(NP, 64), ei, zeros64)
    g4p8 = _tc_d(acc3.reshape(NC, M2, 128), dinvE, b3t, K, dinvE8)

    acc4 = sc16(g4p8.reshape(NP, 16), ei, zeros16)
    g5p8 = _tc_e(acc4.reshape(NC, M8, 128), dinvE8, b4t8)

    acc5 = sc16(g5p8.reshape(NP, 16), ei, zeros16)
    z = _tc_f(acc5.reshape(NC, M8, 128), dinvE8, W5bd8, b5t8, P)
    return z[:N // 8].reshape(N, 2)
